# baseline (speedup 1.0000x reference)
import functools

import jax
import jax.numpy as jnp
from jax import lax
from jax.experimental import pallas as pl
from jax.experimental.pallas import tpu as pltpu
from jax.experimental.pallas import tpu_sc as plsc

_NC = 2
_NS = 16
_NW = _NC * _NS


def _relayout_sc(table_t):
    D, V = table_t.shape
    S = 512
    n_bulk = V // S
    P = S * D // 128
    R = (n_bulk + 1) * P
    steps_w = (n_bulk + _NW - 1) // _NW
    TW = V - n_bulk * S
    TP = (TW + 127) // 128 * 128

    mesh = plsc.VectorSubcoreMesh(core_axis_name="c", subcore_axis_name="s")

    @functools.partial(
        pl.kernel,
        mesh=mesh,
        out_type=jax.ShapeDtypeStruct((R, 128), jnp.float32),
        compiler_params=pltpu.CompilerParams(
            use_tc_tiling_on_sc=True, needs_layout_passes=False),
        scratch_types=[
            pltpu.VMEM((D, S), jnp.float32),
            pltpu.VMEM((D, S), jnp.float32),
            pltpu.VMEM((P, 128), jnp.float32),
            pltpu.SemaphoreType.DMA,
            pltpu.SemaphoreType.DMA,
        ],
    )
    def relayout_kernel(tab_hbm, tail_hbm, out_hbm, in0, in1, ob,
                        si0, si1):
        wid = lax.axis_index("s") * _NC + lax.axis_index("c")
        iota = lax.iota(jnp.int32, 16)
        dvec = (iota, iota + 16)
        zero16 = jnp.zeros((16,), jnp.int32)

        def issue_in(buf, sem, j):
            pltpu.async_copy(tab_hbm.at[:, pl.ds(j * S, S)], buf, sem)

        def wait_in(buf, sem, j):
            pltpu.make_async_copy(
                tab_hbm.at[:, pl.ds(j * S, S)], buf, sem).wait()

        def transpose(buf, n_p):
            @plsc.parallel_loop(0, n_p, unroll=8)
            def p_body(p):
                for q in range(4):
                    vl = zero16 + (4 * p + q)
                    for m in range(2):
                        v = plsc.load_gather(buf, [dvec[m], vl])
                        ob[p, pl.ds(16 * (2 * q + m), 16)] = v

        def out_write(j):
            pltpu.sync_copy(ob, out_hbm.at[pl.ds(j * P, P)])

        j_first = wid * steps_w

        @pl.when(j_first < n_bulk)
        def _():
            issue_in(in0, si0, j_first)

        def step_body(g, carry):
            j0 = wid * steps_w + 2 * g
            j1 = j0 + 1

            @pl.when(j1 < n_bulk)
            def _():
                issue_in(in1, si1, j1)

            @pl.when(j0 < n_bulk)
            def _():
                wait_in(in0, si0, j0)
                transpose(in0, P)
                out_write(j0)

            @pl.when(j0 + 2 < n_bulk)
            def _():
                issue_in(in0, si0, j0 + 2)

            @pl.when(j1 < n_bulk)
            def _():
                wait_in(in1, si1, j1)
                transpose(in1, P)
                out_write(j1)

            return carry

        lax.fori_loop(0, (steps_w + 1) // 2, step_body, 0)

        @pl.when(wid == _NW - 1)
        def _():
            n_pt = TP * D // 128
            for a in range(D // 8):
                pltpu.sync_copy(tail_hbm.at[pl.ds(8 * a, 8)],
                                in0.at[pl.ds(8 * a, 8), pl.ds(0, TP)])
            transpose(in0, n_pt)
            pltpu.sync_copy(ob.at[pl.ds(0, n_pt)],
                            out_hbm.at[pl.ds(n_bulk * P, n_pt)])

    def run(tab_t):
        t0 = n_bulk * S
        tail = jnp.pad(lax.slice(tab_t, (0, t0), (D, V)),
                       ((0, 0), (0, TP - TW)))
        return relayout_kernel(tab_t, tail)

    return run(table_t)


def _pool_sc(x, table):
    B, L = x.shape
    V, D = table.shape
    b_per_w = B // _NW
    c0 = min(L, 128)
    c1 = L - c0
    inv_l = 1.0 / L
    n_vreg = D // 16

    mesh = plsc.VectorSubcoreMesh(core_axis_name="c", subcore_axis_name="s")

    UN = 8
    assert L % UN == 0

    @functools.partial(
        pl.kernel,
        mesh=mesh,
        out_type=jax.ShapeDtypeStruct((B, D), jnp.float32),
        compiler_params=pltpu.CompilerParams(use_tc_tiling_on_sc=False),
        scratch_types=[
            pltpu.VMEM((b_per_w, L), jnp.int32),
            pltpu.VMEM((L, D), jnp.float32),
            pltpu.VMEM((L, D), jnp.float32),
            pltpu.VMEM((b_per_w, D), jnp.float32),
            pltpu.SemaphoreType.DMA,
            pltpu.SemaphoreType.DMA,
        ],
    )
    def pool_kernel(x_hbm, table_hbm, out_hbm, idx_v, rows0_v, rows1_v,
                    pooled_v, sem0, sem1):
        wid = lax.axis_index("s") * _NC + lax.axis_index("c")
        base = wid * b_per_w
        pltpu.sync_copy(x_hbm.at[pl.ds(base, b_per_w)], idx_v)

        def issue(slot_ref, sem, b):
            pltpu.async_copy(
                table_hbm.at[idx_v.at[b, pl.ds(0, c0)]],
                slot_ref.at[pl.ds(0, c0)], sem)
            pltpu.async_copy(
                table_hbm.at[idx_v.at[b, pl.ds(c0, c1)]],
                slot_ref.at[pl.ds(c0, c1)], sem)

        def wait(slot_ref, sem):
            pltpu.make_async_copy(
                table_hbm.at[idx_v.at[0, pl.ds(0, c0)]],
                slot_ref.at[pl.ds(0, c0)], sem).wait()
            pltpu.make_async_copy(
                table_hbm.at[idx_v.at[0, pl.ds(c0, c1)]],
                slot_ref.at[pl.ds(c0, c1)], sem).wait()

        def accumulate(slot_ref, b):
            zero = jnp.zeros((16,), jnp.float32)

            def acc_body(i, accs):
                r = i * UN
                a = [[acc for acc in chain] for chain in accs]
                for k in range(UN):
                    for j in range(n_vreg):
                        a[j][k % 2] = a[j][k % 2] + slot_ref[
                            r + k, pl.ds(j * 16, 16)]
                return tuple(tuple(chain) for chain in a)

            accs = plsc.parallel_loop(
                0, L // UN,
                carry=tuple((zero, zero) for _ in range(n_vreg)))(acc_body)
            for j in range(n_vreg):
                pooled_v[b, pl.ds(j * 16, 16)] = (
                    (accs[j][0] + accs[j][1]) * inv_l)

        issue(rows0_v, sem0, 0)

        def row_body(g, carry):
            b0 = 2 * g
            issue(rows1_v, sem1, b0 + 1)
            wait(rows0_v, sem0)
            accumulate(rows0_v, b0)

            @pl.when(b0 + 2 < b_per_w)
            def _():
                issue(rows0_v, sem0, b0 + 2)

            wait(rows1_v, sem1)
            accumulate(rows1_v, b0 + 1)
            return carry

        lax.fori_loop(0, b_per_w // 2, row_body, 0)
        pltpu.sync_copy(pooled_v, out_hbm.at[pl.ds(base, b_per_w)])

    return pool_kernel(x, table)


def _mlp_tc(pooled, W1, b1, W2, b2, W3, b3):
    B = pooled.shape[0]
    C = W3.shape[1]

    def mlp_kernel(p_ref, w1_ref, b1_ref, w2_ref, b2_ref, w3_ref, b3_ref,
                   o_ref):
        h = jnp.dot(p_ref[...], w1_ref[...],
                    preferred_element_type=jnp.float32) + b1_ref[...]
        h = jnp.maximum(h, 0.0)
        h = jnp.dot(h, w2_ref[...],
                    preferred_element_type=jnp.float32) + b2_ref[...]
        h = jnp.maximum(h, 0.0)
        o_ref[...] = jnp.dot(h, w3_ref[...],
                             preferred_element_type=jnp.float32) + b3_ref[...]

    return pl.pallas_call(
        mlp_kernel,
        out_shape=jax.ShapeDtypeStruct((B, C), jnp.float32),
    )(pooled, W1, b1.reshape(1, -1), W2, b2.reshape(1, -1), W3,
      b3.reshape(1, -1))


def kernel(x, table, W1, b1, W2, b2, W3, b3):
    V, D = table.shape
    packed = _relayout_sc(table.T)
    table_lin = packed.reshape(packed.shape[0] * 128 // D, D)
    pooled = _pool_sc(x, table_lin)
    return _mlp_tc(pooled, W1, b1, W2, b2, W3, b3)


# speedup vs baseline: 1.0009x; 1.0009x over previous
import functools

import jax
import jax.numpy as jnp
from jax import lax
from jax.experimental import pallas as pl
from jax.experimental.pallas import tpu as pltpu
from jax.experimental.pallas import tpu_sc as plsc

_NC = 2
_NS = 16
_NW = _NC * _NS


def _relayout_sc(table_t):
    D, V = table_t.shape
    S = 512
    n_bulk = V // S
    P = S * D // 128
    R = (n_bulk + 1) * P
    steps_w = (n_bulk + _NW - 1) // _NW
    TW = V - n_bulk * S
    TP = (TW + 127) // 128 * 128

    mesh = plsc.VectorSubcoreMesh(core_axis_name="c", subcore_axis_name="s")

    @functools.partial(
        pl.kernel,
        mesh=mesh,
        out_type=jax.ShapeDtypeStruct((R, 128), jnp.float32),
        compiler_params=pltpu.CompilerParams(
            use_tc_tiling_on_sc=True, needs_layout_passes=False),
        scratch_types=[
            pltpu.VMEM((D, S), jnp.float32),
            pltpu.VMEM((D, S), jnp.float32),
            pltpu.VMEM((P, 128), jnp.float32),
            pltpu.SemaphoreType.DMA,
            pltpu.SemaphoreType.DMA,
        ],
    )
    def relayout_kernel(tab_hbm, tail_hbm, out_hbm, in0, in1, ob,
                        si0, si1):
        wid = lax.axis_index("s") * _NC + lax.axis_index("c")
        iota = lax.iota(jnp.int32, 16)
        dvec = (iota, iota + 16)
        zero16 = jnp.zeros((16,), jnp.int32)

        def issue_in(buf, sem, j):
            pltpu.async_copy(tab_hbm.at[:, pl.ds(j * S, S)], buf, sem)

        def wait_in(buf, sem, j):
            pltpu.make_async_copy(
                tab_hbm.at[:, pl.ds(j * S, S)], buf, sem).wait()

        def transpose(buf, n_p):
            @plsc.parallel_loop(0, n_p, unroll=4)
            def p_body(p):
                for q in range(4):
                    vl = zero16 + (4 * p + q)
                    for m in range(2):
                        v = plsc.load_gather(buf, [dvec[m], vl])
                        ob[p, pl.ds(16 * (2 * q + m), 16)] = v

        def out_write(j):
            pltpu.sync_copy(ob, out_hbm.at[pl.ds(j * P, P)])

        j_first = wid * steps_w

        @pl.when(j_first < n_bulk)
        def _():
            issue_in(in0, si0, j_first)

        def step_body(g, carry):
            j0 = wid * steps_w + 2 * g
            j1 = j0 + 1

            @pl.when(j1 < n_bulk)
            def _():
                issue_in(in1, si1, j1)

            @pl.when(j0 < n_bulk)
            def _():
                wait_in(in0, si0, j0)
                transpose(in0, P)
                out_write(j0)

            @pl.when(j0 + 2 < n_bulk)
            def _():
                issue_in(in0, si0, j0 + 2)

            @pl.when(j1 < n_bulk)
            def _():
                wait_in(in1, si1, j1)
                transpose(in1, P)
                out_write(j1)

            return carry

        lax.fori_loop(0, (steps_w + 1) // 2, step_body, 0)

        @pl.when(wid == _NW - 1)
        def _():
            n_pt = TP * D // 128
            for a in range(D // 8):
                pltpu.sync_copy(tail_hbm.at[pl.ds(8 * a, 8)],
                                in0.at[pl.ds(8 * a, 8), pl.ds(0, TP)])
            transpose(in0, n_pt)
            pltpu.sync_copy(ob.at[pl.ds(0, n_pt)],
                            out_hbm.at[pl.ds(n_bulk * P, n_pt)])

    def run(tab_t):
        t0 = n_bulk * S
        tail = jnp.pad(lax.slice(tab_t, (0, t0), (D, V)),
                       ((0, 0), (0, TP - TW)))
        return relayout_kernel(tab_t, tail)

    return run(table_t)


def _pool_sc(x, table):
    B, L = x.shape
    V, D = table.shape
    b_per_w = B // _NW
    c0 = min(L, 128)
    c1 = L - c0
    inv_l = 1.0 / L
    n_vreg = D // 16

    mesh = plsc.VectorSubcoreMesh(core_axis_name="c", subcore_axis_name="s")

    UN = 8
    assert L % UN == 0

    @functools.partial(
        pl.kernel,
        mesh=mesh,
        out_type=jax.ShapeDtypeStruct((B, D), jnp.float32),
        compiler_params=pltpu.CompilerParams(use_tc_tiling_on_sc=False),
        scratch_types=[
            pltpu.VMEM((b_per_w, L), jnp.int32),
            pltpu.VMEM((L, D), jnp.float32),
            pltpu.VMEM((L, D), jnp.float32),
            pltpu.VMEM((b_per_w, D), jnp.float32),
            pltpu.SemaphoreType.DMA,
            pltpu.SemaphoreType.DMA,
        ],
    )
    def pool_kernel(x_hbm, table_hbm, out_hbm, idx_v, rows0_v, rows1_v,
                    pooled_v, sem0, sem1):
        wid = lax.axis_index("s") * _NC + lax.axis_index("c")
        base = wid * b_per_w
        pltpu.sync_copy(x_hbm.at[pl.ds(base, b_per_w)], idx_v)

        def issue(slot_ref, sem, b):
            pltpu.async_copy(
                table_hbm.at[idx_v.at[b, pl.ds(0, c0)]],
                slot_ref.at[pl.ds(0, c0)], sem)
            pltpu.async_copy(
                table_hbm.at[idx_v.at[b, pl.ds(c0, c1)]],
                slot_ref.at[pl.ds(c0, c1)], sem)

        def wait(slot_ref, sem):
            pltpu.make_async_copy(
                table_hbm.at[idx_v.at[0, pl.ds(0, c0)]],
                slot_ref.at[pl.ds(0, c0)], sem).wait()
            pltpu.make_async_copy(
                table_hbm.at[idx_v.at[0, pl.ds(c0, c1)]],
                slot_ref.at[pl.ds(c0, c1)], sem).wait()

        def accumulate(slot_ref, b):
            zero = jnp.zeros((16,), jnp.float32)

            def acc_body(i, accs):
                r = i * UN
                a = [[acc for acc in chain] for chain in accs]
                for k in range(UN):
                    for j in range(n_vreg):
                        a[j][k % 2] = a[j][k % 2] + slot_ref[
                            r + k, pl.ds(j * 16, 16)]
                return tuple(tuple(chain) for chain in a)

            accs = plsc.parallel_loop(
                0, L // UN,
                carry=tuple((zero, zero) for _ in range(n_vreg)))(acc_body)
            for j in range(n_vreg):
                pooled_v[b, pl.ds(j * 16, 16)] = (
                    (accs[j][0] + accs[j][1]) * inv_l)

        issue(rows0_v, sem0, 0)

        def row_body(g, carry):
            b0 = 2 * g
            issue(rows1_v, sem1, b0 + 1)
            wait(rows0_v, sem0)
            accumulate(rows0_v, b0)

            @pl.when(b0 + 2 < b_per_w)
            def _():
                issue(rows0_v, sem0, b0 + 2)

            wait(rows1_v, sem1)
            accumulate(rows1_v, b0 + 1)
            return carry

        lax.fori_loop(0, b_per_w // 2, row_body, 0)
        pltpu.sync_copy(pooled_v, out_hbm.at[pl.ds(base, b_per_w)])

    return pool_kernel(x, table)


def _mlp_tc(pooled, W1, b1, W2, b2, W3, b3):
    B = pooled.shape[0]
    C = W3.shape[1]

    def mlp_kernel(p_ref, w1_ref, b1_ref, w2_ref, b2_ref, w3_ref, b3_ref,
                   o_ref):
        h = jnp.dot(p_ref[...], w1_ref[...],
                    preferred_element_type=jnp.float32) + b1_ref[...]
        h = jnp.maximum(h, 0.0)
        h = jnp.dot(h, w2_ref[...],
                    preferred_element_type=jnp.float32) + b2_ref[...]
        h = jnp.maximum(h, 0.0)
        o_ref[...] = jnp.dot(h, w3_ref[...],
                             preferred_element_type=jnp.float32) + b3_ref[...]

    return pl.pallas_call(
        mlp_kernel,
        out_shape=jax.ShapeDtypeStruct((B, C), jnp.float32),
    )(pooled, W1, b1.reshape(1, -1), W2, b2.reshape(1, -1), W3,
      b3.reshape(1, -1))


def kernel(x, table, W1, b1, W2, b2, W3, b3):
    V, D = table.shape
    packed = _relayout_sc(table.T)
    table_lin = packed.reshape(packed.shape[0] * 128 // D, D)
    pooled = _pool_sc(x, table_lin)
    return _mlp_tc(pooled, W1, b1, W2, b2, W3, b3)
